# baseline (speedup 1.0000x reference)
import functools

import jax
import jax.numpy as jnp
from jax import lax
from jax.experimental import pallas as pl
from jax.experimental.pallas import tpu as pltpu
from jax.experimental.pallas import tpu_sc as plsc

_info = plsc.get_sparse_core_info()
_NC, _NS, _L = _info.num_cores, _info.num_subcores, _info.num_lanes
_NW = _NC * _NS

_TEX_H = 72
_TEX_W = 128
_LANE = 128
_SUB = 8
_SS = 4


def _make_sc_kernel(batch, h, w):
    n_points = batch * h * w
    slabs = _NW // batch
    rows = h // slabs
    cblk = w // _LANE
    grp = _LANE // _L
    chunk = rows * w
    srows = rows // _SS
    sx = srows * 2 * w
    so = srows * w
    mesh = plsc.VectorSubcoreMesh(core_axis_name="c", subcore_axis_name="s")

    @functools.partial(
        pl.kernel,
        mesh=mesh,
        out_type=jax.ShapeDtypeStruct((n_points,), jnp.float32),
        scratch_types=[
            pltpu.VMEM((sx,), jnp.float32),
            pltpu.VMEM((sx,), jnp.float32),
            pltpu.VMEM((_TEX_H * _TEX_W,), jnp.float32),
            pltpu.VMEM((chunk,), jnp.float32),
            pltpu.SemaphoreType.DMA,
            pltpu.SemaphoreType.DMA,
            pltpu.SemaphoreType.DMA,
        ],
        compiler_params=pltpu.CompilerParams(needs_layout_passes=False),
    )
    def sc_kernel(x_hbm, tex_hbm, out_hbm, xb0, xb1, texv, outv, s0, s1, so_sem):
        wid = lax.axis_index("s") * _NC + lax.axis_index("c")
        base = wid * chunk
        xbufs, xsems = (xb0, xb1), (s0, s1)
        in_h = [None, None]
        in_h[0] = pltpu.async_copy(x_hbm.at[pl.ds(2 * base, sx)], xb0, s0)
        in_h[1] = pltpu.async_copy(x_hbm.at[pl.ds(2 * base + sx, sx)], xb1, s1)
        pltpu.sync_copy(tex_hbm, texv)

        out_h = []
        for s in range(_SS):
            xv = xbufs[s % 2]
            in_h[s % 2].wait()

            def row_body(rl, carry, s=s, xv=xv):
                r = s * srows + rl
                xrow = rl * (2 * w)
                orow = (r >> 3) * (cblk * _SUB * _LANE) + (r & 7) * _LANE
                for cb in range(cblk):
                    xb = xrow + cb * (2 * _LANE)
                    ob = orow + cb * (_SUB * _LANE)
                    for k in range(grp):
                        gx = xv[pl.ds(xb + k * _L, _L)]
                        gy = xv[pl.ds(xb + _LANE + k * _L, _L)]
                        ux = gx * 64.0 + 63.5
                        uy = gy * 64.0 + 63.5
                        jx = ux.astype(jnp.int32)
                        jy = uy.astype(jnp.int32)
                        fx = ux - jx.astype(jnp.float32)
                        fy = uy - jy.astype(jnp.float32)
                        f00 = (jy << 7) + jx - (63 * _TEX_W + 63)
                        f10 = f00 + _TEX_W
                        v00 = plsc.load_gather(texv, [f00])
                        v01 = plsc.load_gather(texv, [f00 + 1])
                        v10 = plsc.load_gather(texv, [f10])
                        v11 = plsc.load_gather(texv, [f10 + 1])
                        t0 = v00 + fx * (v01 - v00)
                        t1 = v10 + fx * (v11 - v10)
                        outv[pl.ds(ob + k * _L, _L)] = t0 + fy * (t1 - t0)
                return carry

            lax.fori_loop(0, srows, row_body, 0)
            if s + 2 < _SS:
                in_h[s % 2] = pltpu.async_copy(
                    x_hbm.at[pl.ds(2 * base + (s + 2) * sx, sx)],
                    xbufs[s % 2],
                    xsems[s % 2],
                )
            out_h.append(
                pltpu.async_copy(
                    outv.at[pl.ds(s * so, so)],
                    out_hbm.at[pl.ds(base + s * so, so)],
                    so_sem,
                )
            )
        for hnd in out_h:
            hnd.wait()

    return sc_kernel


def kernel(x, layer1):
    batch, h, w, _ = x.shape
    xflat = (
        x.reshape(batch, h, w // _LANE, _LANE, 2)
        .transpose(0, 1, 2, 4, 3)
        .reshape(-1)
    )
    texp = jnp.pad(layer1[0, 0], ((1, _TEX_H - 65), (1, _TEX_W - 65)))
    out = _make_sc_kernel(batch, h, w)(xflat, texp.reshape(-1))
    return (
        out.reshape(batch, h // _SUB, w // _LANE, _SUB, _LANE)
        .transpose(0, 1, 3, 2, 4)
        .reshape(batch, 1, h, w)
    )


# speedup vs baseline: 1.0445x; 1.0445x over previous
import functools

import jax
import jax.numpy as jnp
from jax import lax
from jax.experimental import pallas as pl
from jax.experimental.pallas import tpu as pltpu
from jax.experimental.pallas import tpu_sc as plsc

_info = plsc.get_sparse_core_info()
_NC, _NS, _L = _info.num_cores, _info.num_subcores, _info.num_lanes
_NW = _NC * _NS

_TEX_H = 72
_TEX_W = 128
_LANE = 128
_SUB = 8


def _make_sc_kernel(batch, h, w):
    n_points = batch * h * w
    slabs = _NW // batch
    rows = h // slabs
    cblk = w // _LANE
    grp = _LANE // _L
    chunk = rows * w
    mesh = plsc.VectorSubcoreMesh(core_axis_name="c", subcore_axis_name="s")

    @functools.partial(
        pl.kernel,
        mesh=mesh,
        out_type=jax.ShapeDtypeStruct((n_points,), jnp.float32),
        scratch_types=[
            pltpu.VMEM((2 * chunk,), jnp.float32),
            pltpu.VMEM((_TEX_H * _TEX_W,), jnp.float32),
            pltpu.VMEM((chunk,), jnp.float32),
        ],
        compiler_params=pltpu.CompilerParams(needs_layout_passes=False),
    )
    def sc_kernel(x_hbm, tex_hbm, out_hbm, xv, texv, outv):
        wid = lax.axis_index("s") * _NC + lax.axis_index("c")
        base = wid * chunk
        pltpu.sync_copy(tex_hbm, texv)
        pltpu.sync_copy(x_hbm.at[pl.ds(2 * base, 2 * chunk)], xv)

        def row_body(r, carry):
            xrow = r * (2 * w)
            orow = (r >> 3) * (cblk * _SUB * _LANE) + (r & 7) * _LANE
            for cb in range(cblk):
                xb = xrow + cb * (2 * _LANE)
                ob = orow + cb * (_SUB * _LANE)
                for k in range(grp):
                    gx = xv[pl.ds(xb + k * _L, _L)]
                    gy = xv[pl.ds(xb + _LANE + k * _L, _L)]
                    ux = gx * 64.0 + 63.5
                    uy = gy * 64.0 + 63.5
                    jx = ux.astype(jnp.int32)
                    jy = uy.astype(jnp.int32)
                    fx = ux - jx.astype(jnp.float32)
                    fy = uy - jy.astype(jnp.float32)
                    f00 = (jy << 7) + jx - (63 * _TEX_W + 63)
                    f10 = f00 + _TEX_W
                    v00 = plsc.load_gather(texv, [f00])
                    v01 = plsc.load_gather(texv, [f00 + 1])
                    v10 = plsc.load_gather(texv, [f10])
                    v11 = plsc.load_gather(texv, [f10 + 1])
                    t0 = v00 + fx * (v01 - v00)
                    t1 = v10 + fx * (v11 - v10)
                    outv[pl.ds(ob + k * _L, _L)] = t0 + fy * (t1 - t0)
            return carry

        lax.fori_loop(0, rows, row_body, 0)
        pltpu.sync_copy(outv, out_hbm.at[pl.ds(base, chunk)])

    return sc_kernel


def kernel(x, layer1):
    batch, h, w, _ = x.shape
    xflat = (
        x.reshape(batch, h, w // _LANE, _LANE, 2)
        .transpose(0, 1, 2, 4, 3)
        .reshape(-1)
    )
    texp = jnp.pad(layer1[0, 0], ((1, _TEX_H - 65), (1, _TEX_W - 65)))
    out = _make_sc_kernel(batch, h, w)(xflat, texp.reshape(-1))
    return (
        out.reshape(batch, h // _SUB, w // _LANE, _SUB, _LANE)
        .transpose(0, 1, 3, 2, 4)
        .reshape(batch, 1, h, w)
    )


# speedup vs baseline: 2.0817x; 1.9929x over previous
import functools

import jax
import jax.numpy as jnp
from jax import lax
from jax.experimental import pallas as pl
from jax.experimental.pallas import tpu as pltpu
from jax.experimental.pallas import tpu_sc as plsc

_info = plsc.get_sparse_core_info()
_NC, _NS, _L = _info.num_cores, _info.num_subcores, _info.num_lanes
_NW = _NC * _NS

_TEX_H = 72
_TEX_W = 128
_LANE = 128
_SUB = 8


def _make_sc_kernel(batch, h, w):
    n_points = batch * h * w
    slabs = _NW // batch
    rows = h // slabs
    cblk = w // _LANE
    grp = _LANE // _L
    chunk = rows * w
    mesh = plsc.VectorSubcoreMesh(core_axis_name="c", subcore_axis_name="s")

    @functools.partial(
        pl.kernel,
        mesh=mesh,
        out_type=jax.ShapeDtypeStruct((n_points,), jnp.float32),
        scratch_types=[
            pltpu.VMEM((2 * chunk,), jnp.float32),
            pltpu.VMEM((_TEX_H * _TEX_W,), jnp.float32),
            pltpu.VMEM((chunk,), jnp.float32),
        ],
        compiler_params=pltpu.CompilerParams(needs_layout_passes=False),
    )
    def sc_kernel(x_hbm, tex_hbm, out_hbm, xv, texv, outv):
        wid = lax.axis_index("s") * _NC + lax.axis_index("c")
        base = wid * chunk
        pltpu.sync_copy(tex_hbm, texv)
        pltpu.sync_copy(x_hbm.at[pl.ds(2 * base, 2 * chunk)], xv)

        @plsc.parallel_loop(0, rows * cblk * grp, unroll=8)
        def grp_body(g):
            r = g >> 5
            cb = (g >> 3) & (cblk - 1)
            k = g & (grp - 1)
            xb = r * (2 * w) + cb * (2 * _LANE) + k * _L
            ob = (
                (r >> 3) * (cblk * _SUB * _LANE)
                + cb * (_SUB * _LANE)
                + (r & 7) * _LANE
                + k * _L
            )
            gx = xv[pl.ds(xb, _L)]
            gy = xv[pl.ds(xb + _LANE, _L)]
            ux = gx * 64.0 + 63.5
            uy = gy * 64.0 + 63.5
            jx = ux.astype(jnp.int32)
            jy = uy.astype(jnp.int32)
            fx = ux - jx.astype(jnp.float32)
            fy = uy - jy.astype(jnp.float32)
            f00 = (jy << 7) + jx - (63 * _TEX_W + 63)
            f10 = f00 + _TEX_W
            v00 = plsc.load_gather(texv, [f00])
            v01 = plsc.load_gather(texv, [f00 + 1])
            v10 = plsc.load_gather(texv, [f10])
            v11 = plsc.load_gather(texv, [f10 + 1])
            t0 = v00 + fx * (v01 - v00)
            t1 = v10 + fx * (v11 - v10)
            outv[pl.ds(ob, _L)] = t0 + fy * (t1 - t0)
        pltpu.sync_copy(outv, out_hbm.at[pl.ds(base, chunk)])

    return sc_kernel


def kernel(x, layer1):
    batch, h, w, _ = x.shape
    xflat = (
        x.reshape(batch, h, w // _LANE, _LANE, 2)
        .transpose(0, 1, 2, 4, 3)
        .reshape(-1)
    )
    texp = jnp.pad(layer1[0, 0], ((1, _TEX_H - 65), (1, _TEX_W - 65)))
    out = _make_sc_kernel(batch, h, w)(xflat, texp.reshape(-1))
    return (
        out.reshape(batch, h // _SUB, w // _LANE, _SUB, _LANE)
        .transpose(0, 1, 3, 2, 4)
        .reshape(batch, 1, h, w)
    )


# speedup vs baseline: 2.1642x; 1.0397x over previous
import functools

import jax
import jax.numpy as jnp
from jax import lax
from jax.experimental import pallas as pl
from jax.experimental.pallas import tpu as pltpu
from jax.experimental.pallas import tpu_sc as plsc

_info = plsc.get_sparse_core_info()
_NC, _NS, _L = _info.num_cores, _info.num_subcores, _info.num_lanes
_NW = _NC * _NS

_TEX_H = 72
_TEX_W = 128
_LANE = 128
_SUB = 8
_SS = 4


def _make_sc_kernel(batch, h, w):
    n_points = batch * h * w
    slabs = _NW // batch
    rows = h // slabs
    cblk = w // _LANE
    grp = _LANE // _L
    chunk = rows * w
    mesh = plsc.VectorSubcoreMesh(core_axis_name="c", subcore_axis_name="s")

    @functools.partial(
        pl.kernel,
        mesh=mesh,
        out_type=jax.ShapeDtypeStruct((n_points,), jnp.float32),
        scratch_types=[
            pltpu.VMEM((2 * chunk // _SS,), jnp.float32),
            pltpu.VMEM((2 * chunk // _SS,), jnp.float32),
            pltpu.VMEM((_TEX_H * _TEX_W,), jnp.float32),
            pltpu.VMEM((chunk,), jnp.float32),
            pltpu.SemaphoreType.DMA,
            pltpu.SemaphoreType.DMA,
            pltpu.SemaphoreType.DMA,
        ],
        compiler_params=pltpu.CompilerParams(needs_layout_passes=False),
    )
    def sc_kernel(x_hbm, tex_hbm, out_hbm, xb0, xb1, texv, outv, se0, se1, seo):
        wid = lax.axis_index("s") * _NC + lax.axis_index("c")
        base = wid * chunk
        sgrp = rows * cblk * grp // _SS
        sx = 2 * chunk // _SS
        sow = chunk // _SS
        srows = rows // _SS
        xbufs, xsems = (xb0, xb1), (se0, se1)
        in_h = [
            pltpu.async_copy(x_hbm.at[pl.ds(2 * base, sx)], xb0, se0),
            pltpu.async_copy(x_hbm.at[pl.ds(2 * base + sx, sx)], xb1, se1),
        ]
        pltpu.sync_copy(tex_hbm, texv)

        out_h = []
        for s in range(_SS):
            xv = xbufs[s % 2]
            in_h[s % 2].wait()

            @plsc.parallel_loop(0, sgrp, unroll=8)
            def grp_body(g, s=s, xv=xv):
                rl = g >> 5
                r = s * srows + rl
                cb = (g >> 3) & (cblk - 1)
                k = g & (grp - 1)
                xb = rl * (2 * w) + cb * (2 * _LANE) + k * _L
                ob = (
                    (r >> 3) * (cblk * _SUB * _LANE)
                    + cb * (_SUB * _LANE)
                    + (r & 7) * _LANE
                    + k * _L
                )
                gx = xv[pl.ds(xb, _L)]
                gy = xv[pl.ds(xb + _LANE, _L)]
                ux = gx * 64.0 + 63.5
                uy = gy * 64.0 + 63.5
                jx = ux.astype(jnp.int32)
                jy = uy.astype(jnp.int32)
                fx = ux - jx.astype(jnp.float32)
                fy = uy - jy.astype(jnp.float32)
                f00 = (jy << 7) + jx - (63 * _TEX_W + 63)
                f10 = f00 + _TEX_W
                v00 = plsc.load_gather(texv, [f00])
                v01 = plsc.load_gather(texv, [f00 + 1])
                v10 = plsc.load_gather(texv, [f10])
                v11 = plsc.load_gather(texv, [f10 + 1])
                t0 = v00 + fx * (v01 - v00)
                t1 = v10 + fx * (v11 - v10)
                outv[pl.ds(ob, _L)] = t0 + fy * (t1 - t0)

            if s + 2 < _SS:
                in_h[s % 2] = pltpu.async_copy(
                    x_hbm.at[pl.ds(2 * base + (s + 2) * sx, sx)],
                    xbufs[s % 2],
                    xsems[s % 2],
                )
            out_h.append(
                pltpu.async_copy(
                    outv.at[pl.ds(s * sow, sow)],
                    out_hbm.at[pl.ds(base + s * sow, sow)],
                    seo,
                )
            )
        for hnd in out_h:
            hnd.wait()

    return sc_kernel


def kernel(x, layer1):
    batch, h, w, _ = x.shape
    xflat = (
        x.reshape(batch, h, w // _LANE, _LANE, 2)
        .transpose(0, 1, 2, 4, 3)
        .reshape(-1)
    )
    texp = jnp.pad(layer1[0, 0], ((1, _TEX_H - 65), (1, _TEX_W - 65)))
    out = _make_sc_kernel(batch, h, w)(xflat, texp.reshape(-1))
    return (
        out.reshape(batch, h // _SUB, w // _LANE, _SUB, _LANE)
        .transpose(0, 1, 3, 2, 4)
        .reshape(batch, 1, h, w)
    )
